# Initial kernel scaffold; baseline (speedup 1.0000x reference)
import jax
import jax.numpy as jnp
from jax.experimental import pallas as pl


def kernel(input, table):
    raise NotImplementedError("write your pallas kernel here")


# speedup vs baseline: 1.4666x; 1.4666x over previous
import functools

import jax
import jax.numpy as jnp
from jax import lax
from jax.experimental import pallas as pl
from jax.experimental.pallas import tpu as pltpu
from jax.experimental.pallas import tpu_sc as plsc

VOCAB = 1000000
EMBED_DIM = 32
BATCH = 4096
SEQ = 200
IGNORE_INDEX = -100

B_TOTAL = BATCH * SEQ
NUM_WORKERS = 32
B_PER_W = B_TOTAL // NUM_WORKERS
CHUNK = 1600
N_CHUNKS = B_PER_W // CHUNK
LANES = 16

_mesh = plsc.VectorSubcoreMesh(core_axis_name="c", subcore_axis_name="s")


@functools.partial(
    pl.kernel,
    mesh=_mesh,
    out_type=jax.ShapeDtypeStruct((B_TOTAL, EMBED_DIM), jnp.float32),
    scratch_types=[
        pltpu.VMEM((CHUNK,), jnp.int32),
        pltpu.VMEM((CHUNK, EMBED_DIM), jnp.float32),
        pltpu.SemaphoreType.DMA,
    ],
    compiler_params=pltpu.CompilerParams(use_tc_tiling_on_sc=False),
)
def _embed_sc(idx_hbm, table_hbm, out_hbm, idx_v, rows_v, sem):
    wid = lax.axis_index("s") * 2 + lax.axis_index("c")
    base = wid * B_PER_W

    def chunk_body(i, carry):
        cbase = base + i * CHUNK
        pltpu.sync_copy(idx_hbm.at[pl.ds(cbase, CHUNK)], idx_v)

        def fix_body(j, c):
            v = idx_v[pl.ds(j * LANES, LANES)]
            idx_v[pl.ds(j * LANES, LANES)] = jnp.where(
                v == IGNORE_INDEX, 1, v
            )
            return c

        lax.fori_loop(0, CHUNK // LANES, fix_body, 0)
        pltpu.async_copy(table_hbm.at[idx_v], rows_v, sem).wait()
        pltpu.sync_copy(rows_v, out_hbm.at[pl.ds(cbase, CHUNK)])
        return carry

    lax.fori_loop(0, N_CHUNKS, chunk_body, 0)


def kernel(input, table):
    idx = input.reshape(-1).astype(jnp.int32)
    out = _embed_sc(idx, table)
    return out.reshape(BATCH, SEQ, EMBED_DIM)


# speedup vs baseline: 1.4897x; 1.0157x over previous
import functools

import jax
import jax.numpy as jnp
from jax import lax
from jax.experimental import pallas as pl
from jax.experimental.pallas import tpu as pltpu
from jax.experimental.pallas import tpu_sc as plsc

VOCAB = 1000000
EMBED_DIM = 32
BATCH = 4096
SEQ = 200
IGNORE_INDEX = -100

B_TOTAL = BATCH * SEQ
NUM_WORKERS = 32
B_PER_W = B_TOTAL // NUM_WORKERS
CHUNK = 1600
N_CHUNKS = B_PER_W // CHUNK
LANES = 16

_mesh = plsc.VectorSubcoreMesh(core_axis_name="c", subcore_axis_name="s")


@functools.partial(
    pl.kernel,
    mesh=_mesh,
    out_type=jax.ShapeDtypeStruct((B_TOTAL, EMBED_DIM), jnp.float32),
    scratch_types=[
        pltpu.VMEM((CHUNK,), jnp.int32),
        pltpu.VMEM((CHUNK,), jnp.int32),
        pltpu.VMEM((CHUNK, EMBED_DIM), jnp.float32),
        pltpu.VMEM((CHUNK, EMBED_DIM), jnp.float32),
        pltpu.SemaphoreType.DMA,
        pltpu.SemaphoreType.DMA,
        pltpu.SemaphoreType.DMA,
        pltpu.SemaphoreType.DMA,
        pltpu.SemaphoreType.DMA,
        pltpu.SemaphoreType.DMA,
    ],
    compiler_params=pltpu.CompilerParams(use_tc_tiling_on_sc=False),
)
def _embed_sc(idx_hbm, table_hbm, out_hbm, idx0, idx1, rows0, rows1,
              si0, si1, sg0, sg1, so0, so1):
    wid = lax.axis_index("s") * 2 + lax.axis_index("c")
    base = wid * B_PER_W
    idx_v = (idx0, idx1)
    rows_v = (rows0, rows1)
    sem_i = (si0, si1)
    sem_g = (sg0, sg1)
    sem_o = (so0, so1)

    def start_idx(i, b):
        pltpu.make_async_copy(idx_hbm.at[pl.ds(base + i * CHUNK, CHUNK)],
                              idx_v[b], sem_i[b]).start()

    def wait_idx(b):
        pltpu.make_async_copy(idx_hbm.at[pl.ds(base, CHUNK)],
                              idx_v[b], sem_i[b]).wait()

    def fix(b):
        def fix_body(j, c):
            v = idx_v[b][pl.ds(j * LANES, LANES)]
            idx_v[b][pl.ds(j * LANES, LANES)] = jnp.where(
                v == IGNORE_INDEX, 1, v)
            return c
        lax.fori_loop(0, CHUNK // LANES, fix_body, 0)

    def start_gather(b):
        pltpu.async_copy(table_hbm.at[idx_v[b]], rows_v[b], sem_g[b])

    def wait_gather(b):
        pltpu.make_async_copy(table_hbm.at[idx_v[b]],
                              rows_v[b], sem_g[b]).wait()

    def start_out(i, b):
        pltpu.make_async_copy(rows_v[b],
                              out_hbm.at[pl.ds(base + i * CHUNK, CHUNK)],
                              sem_o[b]).start()

    def wait_out(b):
        pltpu.make_async_copy(rows_v[b],
                              out_hbm.at[pl.ds(base, CHUNK)], sem_o[b]).wait()

    start_idx(0, 0)
    wait_idx(0)
    fix(0)
    start_gather(0)
    start_idx(1, 1)

    for i in range(N_CHUNKS):
        b = i % 2
        nb = 1 - b
        if i + 1 < N_CHUNKS:
            wait_idx(nb)
            fix(nb)
        wait_gather(b)
        if i + 1 < N_CHUNKS:
            if i >= 1:
                wait_out(nb)
            start_gather(nb)
        start_out(i, b)
        if i + 2 < N_CHUNKS:
            start_idx(i + 2, b)

    wait_out((N_CHUNKS - 1) % 2)
    wait_out(N_CHUNKS % 2)


def kernel(input, table):
    idx = input.reshape(-1).astype(jnp.int32)
    out = _embed_sc(idx, table)
    return out.reshape(BATCH, SEQ, EMBED_DIM)


# speedup vs baseline: 1.4915x; 1.0012x over previous
import functools

import jax
import jax.numpy as jnp
from jax import lax
from jax.experimental import pallas as pl
from jax.experimental.pallas import tpu as pltpu
from jax.experimental.pallas import tpu_sc as plsc

VOCAB = 1000000
EMBED_DIM = 32
BATCH = 4096
SEQ = 200
IGNORE_INDEX = -100

B_TOTAL = BATCH * SEQ
NUM_WORKERS = 32
B_PER_W = B_TOTAL // NUM_WORKERS
CHUNK = 1600
N_CHUNKS = B_PER_W // CHUNK
GATHER_STREAMS = 8
LANES = 16

_mesh = plsc.VectorSubcoreMesh(core_axis_name="c", subcore_axis_name="s")


@functools.partial(
    pl.kernel,
    mesh=_mesh,
    out_type=jax.ShapeDtypeStruct((B_TOTAL, EMBED_DIM), jnp.float32),
    scratch_types=[
        pltpu.VMEM((CHUNK,), jnp.int32),
        pltpu.VMEM((CHUNK,), jnp.int32),
        pltpu.VMEM((CHUNK, EMBED_DIM), jnp.float32),
        pltpu.VMEM((CHUNK, EMBED_DIM), jnp.float32),
        pltpu.SemaphoreType.DMA,
        pltpu.SemaphoreType.DMA,
        pltpu.SemaphoreType.DMA,
        pltpu.SemaphoreType.DMA,
        pltpu.SemaphoreType.DMA,
        pltpu.SemaphoreType.DMA,
    ],
    compiler_params=pltpu.CompilerParams(use_tc_tiling_on_sc=False),
)
def _embed_sc(idx_hbm, table_hbm, out_hbm, idx0, idx1, rows0, rows1,
              si0, si1, sg0, sg1, so0, so1):
    wid = lax.axis_index("s") * 2 + lax.axis_index("c")
    base = wid * B_PER_W
    idx_v = (idx0, idx1)
    rows_v = (rows0, rows1)
    sem_i = (si0, si1)
    sem_g = (sg0, sg1)
    sem_o = (so0, so1)

    def start_idx(i, b):
        pltpu.make_async_copy(idx_hbm.at[pl.ds(base + i * CHUNK, CHUNK)],
                              idx_v[b], sem_i[b]).start()

    def wait_idx(b):
        pltpu.make_async_copy(idx_hbm.at[pl.ds(base, CHUNK)],
                              idx_v[b], sem_i[b]).wait()

    def fix(b):
        def fix_body(j, c):
            v = idx_v[b][pl.ds(j * LANES, LANES)]
            idx_v[b][pl.ds(j * LANES, LANES)] = jnp.where(
                v == IGNORE_INDEX, 1, v)
            return c
        lax.fori_loop(0, CHUNK // LANES, fix_body, 0)

    def start_gather(b):
        sub = CHUNK // GATHER_STREAMS
        for s in range(GATHER_STREAMS):
            pltpu.make_async_copy(
                table_hbm.at[idx_v[b].at[pl.ds(s * sub, sub)]],
                rows_v[b].at[pl.ds(s * sub, sub)],
                sem_g[b]).start()

    def wait_gather(b):
        pltpu.make_async_copy(table_hbm.at[idx_v[b]],
                              rows_v[b], sem_g[b]).wait()

    def start_out(i, b):
        pltpu.make_async_copy(rows_v[b],
                              out_hbm.at[pl.ds(base + i * CHUNK, CHUNK)],
                              sem_o[b]).start()

    def wait_out(b):
        pltpu.make_async_copy(rows_v[b],
                              out_hbm.at[pl.ds(base, CHUNK)], sem_o[b]).wait()

    start_idx(0, 0)
    wait_idx(0)
    fix(0)
    start_gather(0)
    start_idx(1, 1)

    for i in range(N_CHUNKS):
        b = i % 2
        nb = 1 - b
        if i + 1 < N_CHUNKS:
            wait_idx(nb)
            fix(nb)
        wait_gather(b)
        if i + 1 < N_CHUNKS:
            if i >= 1:
                wait_out(nb)
            start_gather(nb)
        start_out(i, b)
        if i + 2 < N_CHUNKS:
            start_idx(i + 2, b)

    wait_out((N_CHUNKS - 1) % 2)
    wait_out(N_CHUNKS % 2)


def kernel(input, table):
    idx = input.reshape(-1).astype(jnp.int32)
    out = _embed_sc(idx, table)
    return out.reshape(BATCH, SEQ, EMBED_DIM)


# speedup vs baseline: 1.8880x; 1.2659x over previous
import functools

import jax
import jax.numpy as jnp
from jax import lax
from jax.experimental import pallas as pl
from jax.experimental.pallas import tpu as pltpu
from jax.experimental.pallas import tpu_sc as plsc

VOCAB = 1000000
EMBED_DIM = 32
BATCH = 4096
SEQ = 200
IGNORE_INDEX = -100

B_TOTAL = BATCH * SEQ
NUM_WORKERS = 32
CHUNK = 1024
QB = BATCH // CHUNK
N_UNITS = SEQ * QB
UNITS_PER_W = N_UNITS // NUM_WORKERS
LANES = 16
TR = EMBED_DIM // 8
TCL = CHUNK // 128
JP = 129

_mesh = plsc.VectorSubcoreMesh(core_axis_name="c", subcore_axis_name="s")


@functools.partial(
    pl.kernel,
    mesh=_mesh,
    out_type=jax.ShapeDtypeStruct((SEQ, TR, BATCH // 128, 8, 128),
                                  jnp.float32),
    scratch_types=[
        pltpu.VMEM((CHUNK,), jnp.int32),
        pltpu.VMEM((CHUNK,), jnp.int32),
        pltpu.VMEM((CHUNK, EMBED_DIM), jnp.float32),
        pltpu.VMEM((CHUNK, EMBED_DIM), jnp.float32),
        pltpu.VMEM((TR, TCL, 8, JP), jnp.float32),
        pltpu.SemaphoreType.DMA,
        pltpu.SemaphoreType.DMA,
        pltpu.SemaphoreType.DMA,
        pltpu.SemaphoreType.DMA,
        pltpu.SemaphoreType.DMA,
    ],
    compiler_params=pltpu.CompilerParams(use_tc_tiling_on_sc=False,
                                         needs_layout_passes=False),
    cost_estimate=pl.CostEstimate(
        flops=0, transcendentals=0, bytes_accessed=213_000_000),
)
def _embed_sc(idx_hbm, table_hbm, out_hbm, idx0, idx1, rows0, rows1, st,
              si0, si1, sg0, sg1, so):
    wid = lax.axis_index("s") * 2 + lax.axis_index("c")
    u0 = wid * UNITS_PER_W
    idx_v = (idx0, idx1)
    rows_v = (rows0, rows1)
    sem_i = (si0, si1)
    sem_g = (sg0, sg1)

    l16 = lax.iota(jnp.int32, LANES)
    lo_tr = l16 // 8
    lo_j8 = l16 % 8
    hi_tr = lo_tr + 2
    ones = jnp.ones((LANES,), jnp.int32)

    def start_idx(k, b):
        pltpu.make_async_copy(idx_hbm.at[pl.ds((u0 + k) * CHUNK, CHUNK)],
                              idx_v[b], sem_i[b]).start()

    def wait_idx(b):
        pltpu.make_async_copy(idx_hbm.at[pl.ds(0, CHUNK)],
                              idx_v[b], sem_i[b]).wait()

    def fix(b):
        def fix_body(j, c):
            v = idx_v[b][pl.ds(j * LANES, LANES)]
            idx_v[b][pl.ds(j * LANES, LANES)] = jnp.where(
                v == IGNORE_INDEX, 1, v)
            return c
        lax.fori_loop(0, CHUNK // LANES, fix_body, 0)

    def start_gather(b):
        pltpu.make_async_copy(table_hbm.at[idx_v[b]],
                              rows_v[b], sem_g[b]).start()

    def wait_gather(b):
        pltpu.make_async_copy(table_hbm.at[idx_v[b]],
                              rows_v[b], sem_g[b]).wait()

    def transpose(b):
        rows = rows_v[b]

        def tr_body(i, c):
            tclv = ones * (i // 128)
            b7v = ones * (i % 128)
            row = ones * i
            lo = plsc.load_gather(rows, [row, l16])
            plsc.store_scatter(st, [lo_tr, tclv, lo_j8, b7v], lo)
            hi = plsc.load_gather(rows, [row, LANES + l16])
            plsc.store_scatter(st, [hi_tr, tclv, lo_j8, b7v], hi)
            return c

        lax.fori_loop(0, CHUNK, tr_body, 0)

    def start_out(k):
        u = u0 + k
        s = u // QB
        qb = u % QB
        for tr in range(TR):
            pltpu.make_async_copy(
                st.at[tr, :, :, pl.ds(0, 128)],
                out_hbm.at[s, tr, pl.ds(qb * TCL, TCL)], so).start()

    def wait_out():
        for tr in range(TR):
            pltpu.make_async_copy(
                st.at[tr, :, :, pl.ds(0, 128)],
                out_hbm.at[0, tr, pl.ds(0, TCL)], so).wait()

    start_idx(0, 0)
    wait_idx(0)
    fix(0)
    start_gather(0)
    start_idx(1, 1)

    def step(k, b):
        nb = 1 - b

        @pl.when(k + 1 < UNITS_PER_W)
        def _():
            wait_idx(nb)
            fix(nb)

        wait_gather(b)

        @pl.when(k + 1 < UNITS_PER_W)
        def _():
            start_gather(nb)

        @pl.when(k >= 1)
        def _():
            wait_out()

        transpose(b)
        start_out(k)

        @pl.when(k + 2 < UNITS_PER_W)
        def _():
            start_idx(k + 2, b)

    def pair(k2, c):
        for b in (0, 1):
            k = k2 * 2 + b

            @pl.when(k < UNITS_PER_W)
            def _():
                step(k, b)
        return c

    lax.fori_loop(0, (UNITS_PER_W + 1) // 2, pair, 0)
    wait_out()


def kernel(input, table):
    idx = input.T.reshape(-1)
    out6 = _embed_sc(idx, table)
    out = out6.transpose(2, 4, 0, 1, 3).reshape(BATCH, SEQ, EMBED_DIM)
    return out
